# baseline (speedup 1.0000x reference)
import jax
import jax.numpy as jnp
from jax import lax
from jax.experimental import pallas as pl
from jax.experimental.pallas import tpu as pltpu
from jax.experimental.pallas import tpu_sc as plsc

N = 10000
D = 128
DP = 144
E = 320000
NEG_SLOPE = 0.2
NC, NS, L = 2, 16, 16
NW = NC * NS
EPW = E // NW
CHUNK = 80
NCHUNK = EPW // CHUNK
COVER = 640
LASTN0 = N - COVER


def _mm_body(x_ref, w_ref, a_ref, h1_ref, s1_ref, h2p_ref):
    x = x_ref[...]
    w = w_ref[...]
    av = a_ref[...]
    dn = (((1,), (1,)), ((), ()))
    h1 = lax.dot_general(x, w[:, :D], dn, preferred_element_type=jnp.float32)
    h2 = lax.dot_general(x, w[:, D:], dn, preferred_element_type=jnp.float32)
    h1_ref[...] = h1
    s1_ref[...] = lax.dot_general(h1, av, dn, preferred_element_type=jnp.float32)
    s2 = lax.dot_general(h2, av, dn, preferred_element_type=jnp.float32)
    ones = jnp.ones((N, 1), jnp.float32)
    zeros = jnp.zeros((N, DP - D - 2), jnp.float32)
    h2p_ref[...] = jnp.concatenate([h2, s2, ones, zeros], axis=1)


_mm_call = pl.pallas_call(
    _mm_body,
    out_shape=[
        jax.ShapeDtypeStruct((N, D), jnp.float32),
        jax.ShapeDtypeStruct((N, 1), jnp.float32),
        jax.ShapeDtypeStruct((N, DP), jnp.float32),
    ],
)


def _sc_body(e0_hbm, e1_hbm, s1_hbm, h2p_hbm, part_hbm,
             s1_v, e0i_v, e1i_v, wv_v, rowsa_v, rowsb_v, acc_sh,
             semi, semg, sems):
    cid = lax.axis_index("c")
    sid = lax.axis_index("s")
    wid = cid * NS + sid
    base = wid * EPW

    rows_bufs = (rowsa_v, rowsb_v)

    pltpu.sync_copy(s1_hbm, s1_v)

    zrow0 = jnp.zeros((L,), jnp.float32)

    def zrow(r, carry):
        for j in range(DP // L):
            rowsa_v[r, pl.ds(j * L, L)] = zrow0
            rowsb_v[r, pl.ds(j * L, L)] = zrow0
        return carry

    lax.fori_loop(0, CHUNK, zrow, 0)

    n0 = jnp.minimum(sid * COVER, LASTN0)
    for k in range(COVER // CHUNK):
        pltpu.sync_copy(rowsa_v, acc_sh.at[pl.ds(n0 + k * CHUNK, CHUNK)])
    plsc.subcore_barrier()

    def i_issue(c, ib):
        pltpu.async_copy(e0_hbm.at[pl.ds(base + c * CHUNK, CHUNK)],
                         e0i_v.at[ib], semi[ib])
        pltpu.async_copy(e1_hbm.at[pl.ds(base + c * CHUNK, CHUNK)],
                         e1i_v.at[ib], semi[ib])

    def i_wait(ib):
        pltpu.make_async_copy(e0_hbm.at[pl.ds(0, CHUNK)],
                              e0i_v.at[ib], semi[ib]).wait()
        pltpu.make_async_copy(e1_hbm.at[pl.ds(0, CHUNK)],
                              e1i_v.at[ib], semi[ib]).wait()

    def g_issue(ib, rb):
        pltpu.async_copy(h2p_hbm.at[e1i_v.at[ib]], rows_bufs[rb], semg[rb])

    def g_wait(rb):
        pltpu.make_async_copy(h2p_hbm.at[pl.ds(0, CHUNK)],
                              rows_bufs[rb], semg[rb]).wait()

    def s_issue(ib, rb):
        pltpu.async_copy(rows_bufs[rb], acc_sh.at[e0i_v.at[ib]],
                         sems[rb], add=True)

    def s_wait(rb):
        pltpu.make_async_copy(rows_bufs[rb], acc_sh.at[pl.ds(0, CHUNK)],
                              sems[rb]).wait()

    def compute_scale(ib, rb):
        rows = rows_bufs[rb]
        col128 = jnp.full((L,), D, jnp.int32)
        for g in range(CHUNK // L):
            i0 = e0i_v[ib, pl.ds(g * L, L)]
            sg1 = plsc.load_gather(s1_v, [i0])
            ridx = lax.iota(jnp.int32, L) + g * L
            s2g = plsc.load_gather(rows, [ridx, col128])
            x = sg1 + s2g
            wv_v[pl.ds(g * L, L)] = jnp.exp(-jnp.maximum(x, NEG_SLOPE * x))

        def srow(g, carry2):
            wgrp = wv_v[pl.ds(g * L, L)]
            for u in range(L):
                i = g * L + u
                wv = wgrp[u]
                for j in range(DP // L):
                    rows[i, pl.ds(j * L, L)] = rows[i, pl.ds(j * L, L)] * wv
            return carry2

        lax.fori_loop(0, CHUNK // L, srow, 0)

    i_issue(0, 0)
    i_issue(1, 1)
    i_wait(0)
    g_issue(0, 0)
    s_issue(0, 1)

    def step(c, off):
        rb = off % 2
        ib = off % 4
        g_wait(rb)
        s_wait((rb + 1) % 2)
        i_wait((ib + 1) % 4)
        g_issue((ib + 1) % 4, (rb + 1) % 2)

        @pl.when(c + 2 < NCHUNK)
        def _():
            i_issue(c + 2, (ib + 2) % 4)

        compute_scale(ib, rb)
        s_issue(ib, rb)

    def quad(it, carry):
        c = 4 * it
        for off in range(4):
            step(c + off, off)
        return carry

    lax.fori_loop(0, NCHUNK // 4, quad, 0)
    g_wait(0)
    s_wait(1)
    compute_scale(0, 0)
    s_issue(0, 0)
    s_wait(0)
    plsc.subcore_barrier()

    pltpu.sync_copy(acc_sh.at[pl.ds(n0, COVER)],
                    part_hbm.at[cid, pl.ds(n0, COVER)])


_sc_call = pl.kernel(
    _sc_body,
    out_type=jax.ShapeDtypeStruct((NC, N, DP), jnp.float32),
    mesh=plsc.VectorSubcoreMesh(core_axis_name="c", subcore_axis_name="s",
                                num_cores=NC, num_subcores=NS),
    compiler_params=pltpu.CompilerParams(use_tc_tiling_on_sc=False,
                                         needs_layout_passes=False),
    scratch_types=[
        pltpu.VMEM((N,), jnp.float32),
        pltpu.VMEM((4, CHUNK), jnp.int32),
        pltpu.VMEM((4, CHUNK), jnp.int32),
        pltpu.VMEM((CHUNK,), jnp.float32),
        pltpu.VMEM((CHUNK, DP), jnp.float32),
        pltpu.VMEM((CHUNK, DP), jnp.float32),
        pltpu.VMEM_SHARED((N, DP), jnp.float32),
        [pltpu.SemaphoreType.DMA] * 4,
        [pltpu.SemaphoreType.DMA] * 2,
        [pltpu.SemaphoreType.DMA] * 2,
    ],
)


def _comb_body(h1_ref, part_ref, o_ref):
    p = part_ref[0] + part_ref[1]
    acc = p[:, :D]
    rs = p[:, D + 1:D + 2]
    denom = jnp.where(rs == 0.0, 1e-12, rs)
    h = (h1_ref[...] * rs + acc) / denom
    o_ref[...] = jnp.where(h > 0.0, h, jnp.exp(h) - 1.0)


_comb_call = pl.pallas_call(
    _comb_body,
    out_shape=jax.ShapeDtypeStruct((N, D), jnp.float32),
)


def kernel(input_, edge, W, a):
    edge = edge.astype(jnp.int32)
    h1, s1, h2p = _mm_call(input_, W, a)
    part = _sc_call(edge[0], edge[1], s1.reshape(N), h2p)
    return _comb_call(h1, part)
